# baseline (speedup 1.0000x reference)
import functools

import jax
import jax.numpy as jnp
from jax import lax
from jax.experimental import pallas as pl
from jax.experimental.pallas import tpu as pltpu
from jax.experimental.pallas import tpu_sc as plsc

_NC = 2
_NS = 16
_K = 500
_Q = 4


def _sc_aggregate(table2d, ei3, z2d, n, w):
    nch = ei3.shape[1]
    ch = nch // (_NC * _NS)
    nx = nch - ch * _NC * _NS
    chm = ch // _Q * _Q
    per = (n // _NS) // 8 * 8
    last = n - (_NS - 1) * per
    mesh = plsc.VectorSubcoreMesh(core_axis_name="c", subcore_axis_name="s")

    @functools.partial(
        pl.kernel,
        out_type=jax.ShapeDtypeStruct((_NC, n, w), jnp.float32),
        mesh=mesh,
        scratch_types=[
            pltpu.VMEM((ch + 1, _K), jnp.int32),
            pltpu.VMEM((ch + 1, _K), jnp.int32),
            pltpu.VMEM((_Q, _K, w), jnp.float32),
            pltpu.SemaphoreType.DMA((_Q,)),
            pltpu.SemaphoreType.DMA((_Q,)),
            pltpu.VMEM_SHARED((n, w), jnp.float32),
        ],
        compiler_params=pltpu.CompilerParams(use_tc_tiling_on_sc=False),
    )
    def agg(table_hbm, ei_hbm, z_hbm, out_hbm, sidx, didx,
            rows, gsem, ssem, acc):
        cid = lax.axis_index("c")
        sid = lax.axis_index("s")
        wid = cid * _NS + sid
        base = sid * per
        tbl = table_hbm
        out = out_hbm
        zr = z_hbm
        src2 = ei_hbm.at[0]
        dst2 = ei_hbm.at[1]

        @pl.when(sid < _NS - 1)
        def _():
            pltpu.sync_copy(zr.at[pl.ds(0, per)], acc.at[pl.ds(base, per)])

        @pl.when(sid == _NS - 1)
        def _():
            pltpu.sync_copy(zr, acc.at[pl.ds(base, last)])

        pltpu.sync_copy(src2.at[pl.ds(wid * ch, ch)], sidx.at[pl.ds(0, ch)])
        pltpu.sync_copy(dst2.at[pl.ds(wid * ch, ch)], didx.at[pl.ds(0, ch)])

        @pl.when(wid < nx)
        def _():
            xrow = _NC * _NS * ch + wid
            pltpu.sync_copy(src2.at[pl.ds(xrow, 1)], sidx.at[pl.ds(ch, 1)])
            pltpu.sync_copy(dst2.at[pl.ds(xrow, 1)], didx.at[pl.ds(ch, 1)])

        plsc.subcore_barrier()

        for b in range(_Q):
            pltpu.async_copy(tbl.at[sidx.at[b]], rows.at[b], gsem.at[b])

        @pl.loop(0, chm, step=_Q)
        def _(j):
            descs = []
            for b in range(_Q):
                pltpu.make_async_copy(tbl.at[sidx.at[j + b]],
                                      rows.at[b], gsem.at[b]).wait()
                descs.append(pltpu.async_copy(
                    rows.at[b], acc.at[didx.at[j + b]], ssem.at[b], add=True))
            for b in range(_Q):
                @pl.when(j + _Q + b < chm)
                def _(b=b):
                    descs[b].wait()
                    pltpu.async_copy(tbl.at[sidx.at[j + _Q + b]],
                                     rows.at[b], gsem.at[b])

        for b in range(_Q):
            pltpu.make_async_copy(rows.at[b], acc.at[didx.at[chm - _Q + b]],
                                  ssem.at[b]).wait()

        @pl.loop(chm, ch)
        def _(j):
            pltpu.sync_copy(tbl.at[sidx.at[j]], rows.at[0])
            pltpu.sync_copy(rows.at[0], acc.at[didx.at[j]], add=True)

        @pl.when(wid < nx)
        def _():
            pltpu.sync_copy(tbl.at[sidx.at[ch]], rows.at[1])
            pltpu.sync_copy(rows.at[1], acc.at[didx.at[ch]], add=True)

        plsc.subcore_barrier()

        @pl.when(sid < _NS - 1)
        def _():
            pltpu.sync_copy(acc.at[pl.ds(base, per)],
                            out.at[cid, pl.ds(base, per)])

        @pl.when(sid == _NS - 1)
        def _():
            pltpu.sync_copy(acc.at[pl.ds(base, last)],
                            out.at[cid, pl.ds(base, last)])

    return agg(table2d, ei3, z2d)


def _pack(v, f):
    n, w = v.shape
    t2 = v.reshape(n // f, f, w)
    lanes = f * w
    li = lax.broadcasted_iota(jnp.int32, (w, lanes), 1)
    ji = lax.broadcasted_iota(jnp.int32, (w, lanes), 0)
    out = None
    for q in range(f):
        sq = (li == q * w + ji).astype(jnp.float32)
        term = jnp.dot(t2[:, q, :], sq, preferred_element_type=jnp.float32)
        out = term if out is None else out + term
    return out


def _tc1_body(x_ref, wl_ref, wr_ref, table_ref, r1_ref):
    dims = (((1,), (1,)), ((), ()))
    t1 = lax.dot_general(x_ref[...], wl_ref[...], dims,
                         preferred_element_type=jnp.float32)
    r1 = lax.dot_general(x_ref[...], wr_ref[...], dims,
                         preferred_element_type=jnp.float32)
    tab = jnp.concatenate([t1, jnp.ones_like(t1)], axis=1)
    table_ref[...] = _pack(tab, 4)
    r1_ref[...] = _pack(r1, 4)


def _tc2_body(acc_ref, r1_ref, b1_ref, h_ref, inv_ref):
    a = acc_ref[...]
    s = a[0] + a[1]
    r1p = r1_ref[...]
    h = 16
    hs, invs = [], []
    for q in range(4):
        inv_q = 1.0 / jnp.maximum(s[:, 32 * q + h:32 * q + h + 1], 1.0)
        h_q = jnp.maximum(s[:, 32 * q:32 * q + h] * inv_q
                          + b1_ref[...][None, :]
                          + r1p[:, h * q:h * (q + 1)], 0.0)
        hs.append(h_q)
        invs.append(jnp.broadcast_to(inv_q, inv_q.shape[:1] + (h,)))
    h_ref[...] = _pack(jnp.concatenate(hs, axis=1), 2)
    inv_ref[...] = _pack(jnp.concatenate(invs, axis=1), 2)


def _bd(w_t, f):
    h, c = w_t.shape
    row = jnp.concatenate([w_t] * f, axis=1)
    big = jnp.concatenate([row] * f, axis=0)
    li = lax.broadcasted_iota(jnp.int32, (f * h, f * c), 0)
    mi = lax.broadcasted_iota(jnp.int32, (f * h, f * c), 1)
    return jnp.where((li // h) == (mi // c), big, 0.0)


def _tc3_body(acc2_ref, h_ref, inv_ref, w2l_ref, w2r_ref, b2_ref, o_ref):
    c, hd = w2l_ref.shape
    a2 = acc2_ref[...]
    meanp = (a2[0] + a2[1]) * inv_ref[...]
    zp = (jnp.dot(meanp, _bd(w2l_ref[...].T, 8),
                  preferred_element_type=jnp.float32)
          + jnp.dot(h_ref[...], _bd(w2r_ref[...].T, 8),
                    preferred_element_type=jnp.float32))
    zs = []
    for u in range(8):
        z = zp[:, c * u:c * (u + 1)] + b2_ref[...][None, :]
        z = z - jnp.max(z, axis=1, keepdims=True)
        zs.append(z - jnp.log(jnp.sum(jnp.exp(z), axis=1, keepdims=True)))
    m = zs[0].shape[0]
    o_ref[...] = jnp.stack(zs, axis=1).reshape(m * 8, c)


def kernel(x, edge_index, W1_l, b1_l, W1_r, W2_l, b2_l, W2_r):
    n, d = x.shape
    h = W1_l.shape[0]
    c = W2_l.shape[0]

    e = edge_index.shape[1]
    ei3 = edge_index.reshape(2, e // _K, _K)
    zrows = n - (_NS - 1) * ((n // _NS) // 8 * 8)
    z1 = jnp.zeros((zrows, 2 * h), jnp.float32)
    z2 = jnp.zeros((zrows, h), jnp.float32)

    table1, r1 = pl.pallas_call(
        _tc1_body,
        out_shape=[
            jax.ShapeDtypeStruct((n // 4, 128), jnp.float32),
            jax.ShapeDtypeStruct((n // 4, 4 * h), jnp.float32),
        ],
    )(x, W1_l, W1_r)
    table1 = table1.reshape(n, 2 * h)

    acc1 = _sc_aggregate(table1, ei3, z1, n, 2 * h)

    hfeat, invc = pl.pallas_call(
        _tc2_body,
        out_shape=[jax.ShapeDtypeStruct((n // 8, 128), jnp.float32),
                   jax.ShapeDtypeStruct((n // 8, 128), jnp.float32)],
    )(acc1.reshape(_NC, n // 4, 128), r1, b1_l)

    acc2 = _sc_aggregate(hfeat.reshape(n, h), ei3, z2, n, h)

    out = pl.pallas_call(
        _tc3_body,
        out_shape=jax.ShapeDtypeStruct((n, c), jnp.float32),
    )(acc2.reshape(_NC, n // 8, 128), hfeat, invc, W2_l, W2_r, b2_l)

    return out


# speedup vs baseline: 1.0152x; 1.0152x over previous
import functools

import jax
import jax.numpy as jnp
from jax import lax
from jax.experimental import pallas as pl
from jax.experimental.pallas import tpu as pltpu
from jax.experimental.pallas import tpu_sc as plsc

_NC = 2
_NS = 16
_K = 128
_Q = 6


def _sc_aggregate(table2d, ei3, z2d, n, w):
    nch = ei3.shape[1]
    ch = nch // (_NC * _NS)
    nx = nch - ch * _NC * _NS
    chm = ch // _Q * _Q
    per = (n // _NS) // 8 * 8
    last = n - (_NS - 1) * per
    mesh = plsc.VectorSubcoreMesh(core_axis_name="c", subcore_axis_name="s")

    @functools.partial(
        pl.kernel,
        out_type=jax.ShapeDtypeStruct((_NC, n, w), jnp.float32),
        mesh=mesh,
        scratch_types=[
            pltpu.VMEM((ch + 1, _K), jnp.int32),
            pltpu.VMEM((ch + 1, _K), jnp.int32),
            pltpu.VMEM((_Q, _K, w), jnp.float32),
            pltpu.SemaphoreType.DMA((_Q,)),
            pltpu.SemaphoreType.DMA((_Q,)),
            pltpu.VMEM_SHARED((n, w), jnp.float32),
        ],
        compiler_params=pltpu.CompilerParams(use_tc_tiling_on_sc=False),
    )
    def agg(table_hbm, ei_hbm, z_hbm, out_hbm, sidx, didx,
            rows, gsem, ssem, acc):
        cid = lax.axis_index("c")
        sid = lax.axis_index("s")
        wid = cid * _NS + sid
        base = sid * per
        tbl = table_hbm
        out = out_hbm
        zr = z_hbm
        src2 = ei_hbm.at[0]
        dst2 = ei_hbm.at[1]

        @pl.when(sid < _NS - 1)
        def _():
            pltpu.sync_copy(zr.at[pl.ds(0, per)], acc.at[pl.ds(base, per)])

        @pl.when(sid == _NS - 1)
        def _():
            pltpu.sync_copy(zr, acc.at[pl.ds(base, last)])

        pltpu.sync_copy(src2.at[pl.ds(wid * ch, ch)], sidx.at[pl.ds(0, ch)])
        pltpu.sync_copy(dst2.at[pl.ds(wid * ch, ch)], didx.at[pl.ds(0, ch)])

        @pl.when(wid < nx)
        def _():
            xrow = _NC * _NS * ch + wid
            pltpu.sync_copy(src2.at[pl.ds(xrow, 1)], sidx.at[pl.ds(ch, 1)])
            pltpu.sync_copy(dst2.at[pl.ds(xrow, 1)], didx.at[pl.ds(ch, 1)])

        plsc.subcore_barrier()

        for b in range(_Q):
            pltpu.async_copy(tbl.at[sidx.at[b]], rows.at[b], gsem.at[b])

        @pl.loop(0, chm, step=_Q)
        def _(j):
            descs = []
            for b in range(_Q):
                pltpu.make_async_copy(tbl.at[sidx.at[j + b]],
                                      rows.at[b], gsem.at[b]).wait()
                descs.append(pltpu.async_copy(
                    rows.at[b], acc.at[didx.at[j + b]], ssem.at[b], add=True))
            for b in range(_Q):
                @pl.when(j + _Q + b < chm)
                def _(b=b):
                    descs[b].wait()
                    pltpu.async_copy(tbl.at[sidx.at[j + _Q + b]],
                                     rows.at[b], gsem.at[b])

        for b in range(_Q):
            pltpu.make_async_copy(rows.at[b], acc.at[didx.at[chm - _Q + b]],
                                  ssem.at[b]).wait()

        @pl.loop(chm, ch)
        def _(j):
            pltpu.sync_copy(tbl.at[sidx.at[j]], rows.at[0])
            pltpu.sync_copy(rows.at[0], acc.at[didx.at[j]], add=True)

        @pl.when(wid < nx)
        def _():
            pltpu.sync_copy(tbl.at[sidx.at[ch]], rows.at[1])
            pltpu.sync_copy(rows.at[1], acc.at[didx.at[ch]], add=True)

        plsc.subcore_barrier()

        @pl.when(sid < _NS - 1)
        def _():
            pltpu.sync_copy(acc.at[pl.ds(base, per)],
                            out.at[cid, pl.ds(base, per)])

        @pl.when(sid == _NS - 1)
        def _():
            pltpu.sync_copy(acc.at[pl.ds(base, last)],
                            out.at[cid, pl.ds(base, last)])

    return agg(table2d, ei3, z2d)


def _pack(v, f):
    n, w = v.shape
    t2 = v.reshape(n // f, f, w)
    lanes = f * w
    li = lax.broadcasted_iota(jnp.int32, (w, lanes), 1)
    ji = lax.broadcasted_iota(jnp.int32, (w, lanes), 0)
    out = None
    for q in range(f):
        sq = (li == q * w + ji).astype(jnp.float32)
        term = jnp.dot(t2[:, q, :], sq, preferred_element_type=jnp.float32)
        out = term if out is None else out + term
    return out


def _tc1_body(x_ref, wlt_ref, wrt_ref, table_ref, r1_ref):
    d, h = wlt_ref.shape
    n = x_ref.shape[0]
    x4 = x_ref[...].reshape(n // 4, 4, d)
    wlt = wlt_ref[...]
    wrt = wrt_ref[...]
    zg = jnp.zeros((d, 2 * h), jnp.float32)
    out = None
    for q in range(4):
        blocks = [zg] * q + [jnp.concatenate([wlt, jnp.zeros_like(wlt)], axis=1)]             + [zg] * (3 - q) + [jnp.zeros((d, h * q), jnp.float32), wrt,
                                jnp.zeros((d, h * (3 - q)), jnp.float32)]
        wq = jnp.concatenate([b for b in blocks if b.shape[1]], axis=1)
        term = jnp.dot(x4[:, q, :], wq, preferred_element_type=jnp.float32)
        out = term if out is None else out + term
    li = lax.broadcasted_iota(jnp.int32, (n // 4, 8 * h), 1)
    ones_mask = ((li % (2 * h)) >= h).astype(jnp.float32)
    table_ref[...] = out[:, :8 * h] + ones_mask
    r1_ref[...] = out[:, 8 * h:]


def _tc2_body(acc_ref, r1_ref, b1_ref, h_ref, inv_ref):
    a = acc_ref[...]
    s = a[0] + a[1]
    r1p = r1_ref[...]
    h = 16
    hs, invs = [], []
    for q in range(4):
        inv_q = 1.0 / jnp.maximum(s[:, 32 * q + h:32 * q + h + 1], 1.0)
        h_q = jnp.maximum(s[:, 32 * q:32 * q + h] * inv_q
                          + b1_ref[...][None, :]
                          + r1p[:, h * q:h * (q + 1)], 0.0)
        hs.append(h_q)
        invs.append(jnp.broadcast_to(inv_q, inv_q.shape[:1] + (h,)))
    h_ref[...] = _pack(jnp.concatenate(hs, axis=1), 2)
    inv_ref[...] = _pack(jnp.concatenate(invs, axis=1), 2)


def _bd(w_t, f):
    h, c = w_t.shape
    row = jnp.concatenate([w_t] * f, axis=1)
    big = jnp.concatenate([row] * f, axis=0)
    li = lax.broadcasted_iota(jnp.int32, (f * h, f * c), 0)
    mi = lax.broadcasted_iota(jnp.int32, (f * h, f * c), 1)
    return jnp.where((li // h) == (mi // c), big, 0.0)


def _tc3_body(acc2_ref, h_ref, inv_ref, w2l_ref, w2r_ref, b2_ref, o_ref):
    c, hd = w2l_ref.shape
    a2 = acc2_ref[...]
    meanp = (a2[0] + a2[1]) * inv_ref[...]
    zp = (jnp.dot(meanp, _bd(w2l_ref[...].T, 8),
                  preferred_element_type=jnp.float32)
          + jnp.dot(h_ref[...], _bd(w2r_ref[...].T, 8),
                    preferred_element_type=jnp.float32))
    zs = []
    for u in range(8):
        z = zp[:, c * u:c * (u + 1)] + b2_ref[...][None, :]
        z = z - jnp.max(z, axis=1, keepdims=True)
        zs.append(z - jnp.log(jnp.sum(jnp.exp(z), axis=1, keepdims=True)))
    m = zs[0].shape[0]
    o_ref[...] = jnp.stack(zs, axis=1).reshape(m * 8, c)


def kernel(x, edge_index, W1_l, b1_l, W1_r, W2_l, b2_l, W2_r):
    n, d = x.shape
    h = W1_l.shape[0]
    c = W2_l.shape[0]

    e = edge_index.shape[1]
    ei3 = edge_index.reshape(2, e // _K, _K)
    zrows = n - (_NS - 1) * ((n // _NS) // 8 * 8)
    z1 = jnp.zeros((zrows, 2 * h), jnp.float32)
    z2 = jnp.zeros((zrows, h), jnp.float32)

    table1, r1 = pl.pallas_call(
        _tc1_body,
        out_shape=[
            jax.ShapeDtypeStruct((n // 4, 128), jnp.float32),
            jax.ShapeDtypeStruct((n // 4, 4 * h), jnp.float32),
        ],
    )(x, W1_l.T, W1_r.T)
    table1 = table1.reshape(n, 2 * h)

    acc1 = _sc_aggregate(table1, ei3, z1, n, 2 * h)

    hfeat, invc = pl.pallas_call(
        _tc2_body,
        out_shape=[jax.ShapeDtypeStruct((n // 8, 128), jnp.float32),
                   jax.ShapeDtypeStruct((n // 8, 128), jnp.float32)],
    )(acc1.reshape(_NC, n // 4, 128), r1, b1_l)

    acc2 = _sc_aggregate(hfeat.reshape(n, h), ei3, z2, n, h)

    out = pl.pallas_call(
        _tc3_body,
        out_shape=jax.ShapeDtypeStruct((n, c), jnp.float32),
    )(acc2.reshape(_NC, n // 8, 128), hfeat, invc, W2_l, W2_r, b2_l)

    return out
